# baseline (speedup 1.0000x reference)
import functools

import jax
import jax.numpy as jnp
from jax import lax
from jax.experimental import pallas as pl
from jax.experimental.pallas import tpu as pltpu
from jax.experimental.pallas import tpu_sc as plsc

VOCAB = 1000000
EMBED = 32
BATCH = 16384
FIELDS = 26

B = BATCH * FIELDS
NC = 2
NS = 16
NW = NC * NS
B_PER_W = B // NW
CHUNK = 1664
N_GROUPS = B_PER_W // CHUNK

_mesh = plsc.VectorSubcoreMesh(core_axis_name="c", subcore_axis_name="s")


@functools.partial(
    pl.kernel,
    mesh=_mesh,
    out_type=jax.ShapeDtypeStruct((B, EMBED), jnp.float32),
    scratch_types=[
        pltpu.VMEM((B_PER_W,), jnp.int32),
        pltpu.VMEM((CHUNK, EMBED), jnp.float32),
        pltpu.VMEM((CHUNK, EMBED), jnp.float32),
        pltpu.SemaphoreType.DMA,
        pltpu.SemaphoreType.DMA,
        pltpu.SemaphoreType.DMA,
        pltpu.SemaphoreType.DMA,
    ],
    compiler_params=pltpu.CompilerParams(use_tc_tiling_on_sc=False),
)
def _sc_gather(idx_hbm, table_hbm, out_hbm, idx_v, rows0, rows1, g0, g1, s0, s1):
    wid = lax.axis_index("s") * NC + lax.axis_index("c")
    base = wid * B_PER_W

    bufs = (rows0, rows1)
    gsems = (g0, g1)
    ssems = (s0, s1)

    pltpu.sync_copy(idx_hbm.at[pl.ds(base, B_PER_W)], idx_v)

    def gather(g):
        return pltpu.async_copy(
            table_hbm.at[idx_v.at[pl.ds(g * CHUNK, CHUNK)]],
            bufs[g % 2],
            gsems[g % 2],
        )

    def store(g):
        return pltpu.async_copy(
            bufs[g % 2],
            out_hbm.at[pl.ds(base + g * CHUNK, CHUNK)],
            ssems[g % 2],
        )

    pend_g = gather(0)
    pend_s = [None, None]
    for g in range(N_GROUPS):
        pend_g.wait()
        if g + 1 < N_GROUPS:
            if pend_s[(g + 1) % 2] is not None:
                pend_s[(g + 1) % 2].wait()
            pend_g = gather(g + 1)
        pend_s[g % 2] = store(g)
    pend_s[0].wait()
    pend_s[1].wait()


def kernel(features, table):
    idx = features.astype(jnp.int32).reshape(B)
    out = _sc_gather(idx, table)
    return out.reshape(BATCH, FIELDS, EMBED)


# speedup vs baseline: 1.0049x; 1.0049x over previous
import functools

import jax
import jax.numpy as jnp
from jax import lax
from jax.experimental import pallas as pl
from jax.experimental.pallas import tpu as pltpu
from jax.experimental.pallas import tpu_sc as plsc

VOCAB = 1000000
EMBED = 32
BATCH = 16384
FIELDS = 26

B = BATCH * FIELDS
NC = 2
NS = 16
NW = NC * NS
B_PER_W = B // NW
CHUNK = 832
N_GROUPS = B_PER_W // CHUNK
NBUF = 4
INFLIGHT = 3

_mesh = plsc.VectorSubcoreMesh(core_axis_name="c", subcore_axis_name="s")


@functools.partial(
    pl.kernel,
    mesh=_mesh,
    out_type=jax.ShapeDtypeStruct((B, EMBED), jnp.float32),
    scratch_types=(
        [pltpu.VMEM((B_PER_W,), jnp.int32)]
        + [pltpu.VMEM((CHUNK, EMBED), jnp.float32) for _ in range(NBUF)]
        + [pltpu.SemaphoreType.DMA for _ in range(2 * NBUF)]
    ),
    compiler_params=pltpu.CompilerParams(use_tc_tiling_on_sc=False),
)
def _sc_gather(idx_hbm, table_hbm, out_hbm, idx_v, *bufs_and_sems):
    wid = lax.axis_index("s") * NC + lax.axis_index("c")
    base = wid * B_PER_W

    bufs = bufs_and_sems[:NBUF]
    gsems = bufs_and_sems[NBUF : 2 * NBUF]
    ssems = bufs_and_sems[2 * NBUF :]

    pltpu.sync_copy(idx_hbm.at[pl.ds(base, B_PER_W)], idx_v)

    def gather(g):
        return pltpu.async_copy(
            table_hbm.at[idx_v.at[pl.ds(g * CHUNK, CHUNK)]],
            bufs[g % NBUF],
            gsems[g % NBUF],
        )

    def store(g):
        return pltpu.async_copy(
            bufs[g % NBUF],
            out_hbm.at[pl.ds(base + g * CHUNK, CHUNK)],
            ssems[g % NBUF],
        )

    pend_g = [None] * N_GROUPS
    pend_s = [None] * N_GROUPS
    for g in range(min(INFLIGHT, N_GROUPS)):
        pend_g[g] = gather(g)
    for g in range(N_GROUPS):
        pend_g[g].wait()
        nxt = g + INFLIGHT
        if nxt < N_GROUPS:
            if nxt - NBUF >= 0 and pend_s[nxt - NBUF] is not None:
                pend_s[nxt - NBUF].wait()
            pend_g[nxt] = gather(nxt)
        pend_s[g] = store(g)
    for g in range(max(0, N_GROUPS - NBUF), N_GROUPS):
        if pend_s[g] is not None:
            pend_s[g].wait()


def kernel(features, table):
    idx = features.astype(jnp.int32).reshape(B)
    out = _sc_gather(idx, table)
    return out.reshape(BATCH, FIELDS, EMBED)


# speedup vs baseline: 1.0057x; 1.0008x over previous
import functools

import jax
import jax.numpy as jnp
from jax import lax
from jax.experimental import pallas as pl
from jax.experimental.pallas import tpu as pltpu
from jax.experimental.pallas import tpu_sc as plsc

VOCAB = 1000000
EMBED = 32
BATCH = 16384
FIELDS = 26

B = BATCH * FIELDS
NC = 2
NS = 16
NW = NC * NS
B_PER_W = B // NW
CHUNK = 832
N_GROUPS = B_PER_W // CHUNK
NBUF = 4
INFLIGHT = 3

TCOL = 512


def _pre_body(x_ref, o_ref):
    x = x_ref[...]
    o_ref[...] = jnp.transpose(x, (1, 0)).reshape(TCOL // 4, 128)


_pre_call = pl.pallas_call(
    _pre_body,
    grid=(VOCAB // TCOL,),
    in_specs=[pl.BlockSpec((EMBED, TCOL), lambda i: (0, i))],
    out_specs=pl.BlockSpec((TCOL // 4, 128), lambda i: (i, 0)),
    out_shape=jax.ShapeDtypeStruct((VOCAB // 4, 128), jnp.float32),
)

EB = BATCH // 128


def _post_body(x_ref, o_ref):
    x = x_ref[...].reshape(128, FIELDS, EMBED)
    o_ref[...] = jnp.transpose(x, (1, 2, 0)).reshape(FIELDS, 4, 1, 8, 128)


_post_call = pl.pallas_call(
    _post_body,
    grid=(EB,),
    in_specs=[pl.BlockSpec((128, FIELDS * EMBED), lambda bt: (bt, 0))],
    out_specs=pl.BlockSpec((FIELDS, 4, 1, 8, 128), lambda bt: (0, 0, bt, 0, 0)),
    out_shape=jax.ShapeDtypeStruct((FIELDS, 4, 128, 8, 128), jnp.float32),
)

_mesh = plsc.VectorSubcoreMesh(core_axis_name="c", subcore_axis_name="s")


@functools.partial(
    pl.kernel,
    mesh=_mesh,
    out_type=jax.ShapeDtypeStruct((B, EMBED), jnp.float32),
    scratch_types=(
        [pltpu.VMEM((B_PER_W,), jnp.int32)]
        + [pltpu.VMEM((CHUNK, EMBED), jnp.float32) for _ in range(NBUF)]
        + [pltpu.SemaphoreType.DMA for _ in range(2 * NBUF)]
    ),
    compiler_params=pltpu.CompilerParams(use_tc_tiling_on_sc=False),
)
def _sc_gather(idx_hbm, table_hbm, out_hbm, idx_v, *bufs_and_sems):
    wid = lax.axis_index("s") * NC + lax.axis_index("c")
    base = wid * B_PER_W

    bufs = bufs_and_sems[:NBUF]
    gsems = bufs_and_sems[NBUF : 2 * NBUF]
    ssems = bufs_and_sems[2 * NBUF :]

    pltpu.sync_copy(idx_hbm.at[pl.ds(base, B_PER_W)], idx_v)

    def gather(g):
        return pltpu.async_copy(
            table_hbm.at[idx_v.at[pl.ds(g * CHUNK, CHUNK)]],
            bufs[g % NBUF],
            gsems[g % NBUF],
        )

    def store(g):
        return pltpu.async_copy(
            bufs[g % NBUF],
            out_hbm.at[pl.ds(base + g * CHUNK, CHUNK)],
            ssems[g % NBUF],
        )

    pend_g = [None] * N_GROUPS
    pend_s = [None] * N_GROUPS
    for g in range(min(INFLIGHT, N_GROUPS)):
        pend_g[g] = gather(g)
    for g in range(N_GROUPS):
        pend_g[g].wait()
        nxt = g + INFLIGHT
        if nxt < N_GROUPS:
            if nxt - NBUF >= 0 and pend_s[nxt - NBUF] is not None:
                pend_s[nxt - NBUF].wait()
            pend_g[nxt] = gather(nxt)
        pend_s[g] = store(g)
    for g in range(max(0, N_GROUPS - NBUF), N_GROUPS):
        if pend_s[g] is not None:
            pend_s[g].wait()


def kernel(features, table):
    idx = features.astype(jnp.int32).reshape(B)
    flat = _sc_gather(idx, table)
    return flat.reshape(BATCH, FIELDS, EMBED)
